# baseline (speedup 1.0000x reference)
import functools

import jax
import jax.numpy as jnp
from jax import lax
from jax.experimental import pallas as pl
from jax.experimental.pallas import tpu as pltpu
from jax.experimental.pallas import tpu_sc as plsc

B = 16
L = 256
T = 2048
D = 256
H = 256

NC = 2
NS = 16
NW = NC * NS
CHUNK = 128
NQ = 4
LQ = L // NQ
SENTS = LQ // 2
WIN = 256
META = 16 + 2 * SENTS

CL = 16
NBLK = LQ // CL


def _sc_body(pv_hbm, sll_hbm, sidx_hbm, wemb_hbm, out_hbm,
             len_v, idx_v, sidx_v, rows_v, outbuf_v, sem0, sem1, q):
    c = lax.axis_index("c")
    s = lax.axis_index("s")
    w = c * NS + s
    b = w // 2
    hh = w % 2

    woff = pl.multiple_of(b * T, 8)
    pltpu.async_copy(pv_hbm.at[pl.ds(woff, T)], idx_v, sem1)
    loff = pl.multiple_of(b * L, 8)
    pltpu.sync_copy(sll_hbm.at[pl.ds(loff, L)], len_v)
    pltpu.sync_copy(sidx_hbm.at[w], sidx_v.at[0])

    g0 = q * (2 * SENTS) + hh * SENTS
    nblk0 = g0 // 16

    def blk_sum(i, tot):
        v = len_v[pl.ds(i * 16, 16)]
        for lane in range(16):
            tot = tot + v[lane]
        return tot

    prefix = lax.fori_loop(0, nblk0, blk_sum, jnp.int32(0))

    v0 = len_v[pl.ds(g0, 16)]
    v1 = len_v[pl.ds(g0 + 16, 16)]
    awstart = pl.multiple_of((prefix // 8) * 8, 8)
    lns = [v0[i] for i in range(16)] + [v1[i] for i in range(16)]
    css = []
    stt = prefix - awstart
    for ln_i in lns:
        css.append(stt)
        stt = stt + ln_i
    span = stt
    nch = (span + (CHUNK - 1)) // CHUNK
    pltpu.make_async_copy(pv_hbm.at[pl.ds(woff, T)], idx_v, sem1).wait()

    @pl.when(nch >= 1)
    def _():
        pltpu.async_copy(wemb_hbm.at[idx_v.at[pl.ds(awstart, CHUNK)]],
                         rows_v.at[pl.ds(0, CHUNK)], sem0)

    @pl.when(nch >= 2)
    def _():
        pltpu.async_copy(
            wemb_hbm.at[idx_v.at[pl.ds(awstart + CHUNK, CHUNK)]],
            rows_v.at[pl.ds(CHUNK, CHUNK)], sem1)

    @pl.when(nch >= 1)
    def _():
        pltpu.make_async_copy(wemb_hbm.at[idx_v.at[pl.ds(awstart, CHUNK)]],
                              rows_v.at[pl.ds(0, CHUNK)], sem0).wait()

    @pl.when(nch >= 2)
    def _():
        pltpu.make_async_copy(
            wemb_hbm.at[idx_v.at[pl.ds(awstart + CHUNK, CHUNK)]],
            rows_v.at[pl.ds(CHUNK, CHUNK)], sem1).wait()

    for si in range(2 * 16):
        cs = css[si]
        ln = lns[si]
        acc = [jnp.zeros((16,), jnp.float32) for _ in range(D // 16)]
        for k in range(SLOT_K):
            keep = k < ln
            for cg in range(D // 16):
                v = rows_v[cs + k, pl.ds(cg * 16, 16)]
                acc[cg] = acc[cg] + jnp.where(keep, v, 0.0)
        for cg in range(D // 16):
            outbuf_v[si, pl.ds(cg * 16, 16)] = acc[cg]

    pltpu.async_copy(outbuf_v, out_hbm.at[sidx_v.at[0]], sem0).wait()


SLOT_K = 8


def _segment_sum_sc(q, pv, sll, sidx, W_emb):
    mesh = plsc.VectorSubcoreMesh(core_axis_name="c", subcore_axis_name="s")
    fn = pl.kernel(
        functools.partial(_sc_body, q=q),
        out_type=jax.ShapeDtypeStruct((LQ * B, D), jnp.float32),
        mesh=mesh,
        scratch_types=[
            pltpu.VMEM((L,), jnp.int32),
            pltpu.VMEM((T,), jnp.int32),
            pltpu.VMEM((1, SENTS), jnp.int32),
            pltpu.VMEM((WIN, D), jnp.float32),
            pltpu.VMEM((SENTS, D), jnp.float32),
            pltpu.SemaphoreType.DMA,
            pltpu.SemaphoreType.DMA,
        ],
    )
    return fn(pv, sll, sidx, W_emb)


DN = (((1,), (0,)), ((), ()))


def _lstm_body(toff, maxlen_ref, x_ref, wih_ref, whh_ref, bias_ref, len_ref,
               hin_ref, cin_ref, *rest):
    if len(rest) == 7:
        obuf_ref, out_ref, hout_ref, cout_ref, h_ref, c_ref, gx_ref = rest
    else:
        out_ref, hout_ref, cout_ref, h_ref, c_ref, gx_ref = rest
    blk = pl.program_id(0)

    @pl.when(blk == 0)
    def _():
        h_ref[...] = hin_ref[...]
        c_ref[...] = cin_ref[...]

    active = toff + blk * CL < maxlen_ref[0]

    @pl.when(active)
    def _():
        x2 = x_ref[...].reshape(CL * B, D).astype(jnp.bfloat16)
        gx_ref[...] = (
            lax.dot_general(x2, wih_ref[...], DN,
                            preferred_element_type=jnp.float32)
            + bias_ref[...]
        )

        lens = len_ref[...][:, 0:1]

        def step(j, carry):
            h, c = carry
            t = toff + blk * CL + j
            gates = gx_ref[pl.ds(j * B, B)] + lax.dot_general(
                h.astype(jnp.bfloat16), whh_ref[...], DN,
                preferred_element_type=jnp.float32)
            ii = jax.nn.sigmoid(gates[:, 0:H])
            ff = jax.nn.sigmoid(gates[:, H:2 * H])
            gg = jnp.tanh(gates[:, 2 * H:3 * H])
            oo = jax.nn.sigmoid(gates[:, 3 * H:4 * H])
            c_new = ff * c + ii * gg
            h_new = oo * jnp.tanh(c_new)
            mask = t < lens
            out_ref[j] = jnp.where(mask, h_new, 0.0)
            return (jnp.where(mask, h_new, h), jnp.where(mask, c_new, c))

        hh, cc = lax.fori_loop(0, CL, step, (h_ref[...], c_ref[...]),
                               unroll=16)
        h_ref[...] = hh
        c_ref[...] = cc

    @pl.when(jnp.logical_not(active))
    def _():
        out_ref[...] = jnp.zeros_like(out_ref)

    @pl.when(blk == NBLK - 1)
    def _():
        hout_ref[...] = h_ref[...]
        cout_ref[...] = c_ref[...]


def _lstm_tc(toff, maxlen, x, wih, whh, bias, lens, h0, c0, obuf):
    qbase = toff // CL
    in_specs = [
        pl.BlockSpec(memory_space=pltpu.SMEM),
        pl.BlockSpec((CL, B, D), lambda i: (i, 0, 0)),
        pl.BlockSpec((D, 4 * H), lambda i: (0, 0)),
        pl.BlockSpec((H, 4 * H), lambda i: (0, 0)),
        pl.BlockSpec((1, 4 * H), lambda i: (0, 0)),
        pl.BlockSpec((B, 128), lambda i: (0, 0)),
        pl.BlockSpec((B, H), lambda i: (0, 0)),
        pl.BlockSpec((B, H), lambda i: (0, 0)),
    ]
    args = [maxlen, x, wih, whh, bias, lens, h0, c0]
    aliases = {}
    if obuf is not None:
        in_specs.append(pl.BlockSpec(memory_space=pltpu.MemorySpace.HBM))
        args.append(obuf)
        aliases = {8: 0}
    return pl.pallas_call(
        functools.partial(_lstm_body, toff),
        grid=(NBLK,),
        in_specs=in_specs,
        out_specs=[
            pl.BlockSpec((CL, B, H), lambda i: (qbase + i, 0, 0)),
            pl.BlockSpec((B, H), lambda i: (0, 0)),
            pl.BlockSpec((B, H), lambda i: (0, 0)),
        ],
        out_shape=[
            jax.ShapeDtypeStruct((L, B, H), jnp.float32),
            jax.ShapeDtypeStruct((B, H), jnp.float32),
            jax.ShapeDtypeStruct((B, H), jnp.float32),
        ],
        scratch_shapes=[
            pltpu.VMEM((B, H), jnp.float32),
            pltpu.VMEM((B, H), jnp.float32),
            pltpu.VMEM((CL * B, 4 * H), jnp.float32),
        ],
        input_output_aliases=aliases,
    )(*args)


def kernel(paragraph_variable, sentence_length_list, paragh_length_list,
           max_no_lines, W_emb, W_ih, W_hh, b_ih, b_hh):
    sll = sentence_length_list.astype(jnp.int32).reshape(B * L)

    hh_arr = jnp.arange(NW, dtype=jnp.int32) % 2
    b_of_w = jnp.arange(NW, dtype=jnp.int32) // 2
    j_arr = jnp.arange(SENTS, dtype=jnp.int32)
    sidx = (hh_arr[:, None] * SENTS + j_arr[None, :]) * B + b_of_w[:, None]

    pv = paragraph_variable.astype(jnp.int32).reshape(B * T)

    wih = W_ih.T.astype(jnp.bfloat16)
    whh = W_hh.T.astype(jnp.bfloat16)
    bias = (b_ih + b_hh).reshape(1, 4 * H)
    lens = jnp.broadcast_to(
        paragh_length_list.astype(jnp.int32)[:, None], (B, 128))
    maxlen = paragh_length_list.astype(jnp.int32)[:1]

    xs = [
        _segment_sum_sc(q, pv, sll, sidx, W_emb).reshape(LQ, B, D)
        for q in range(NQ)
    ]

    h = c = jnp.zeros((B, H), jnp.float32)
    out = None
    for q in range(NQ):
        out, h, c = _lstm_tc(q * LQ, maxlen, xs[q], wih, whh, bias,
                             lens, h, c, out)
    return out


# speedup vs baseline: 1.1848x; 1.1848x over previous
import functools

import jax
import jax.numpy as jnp
from jax import lax
from jax.experimental import pallas as pl
from jax.experimental.pallas import tpu as pltpu
from jax.experimental.pallas import tpu_sc as plsc

B = 16
L = 256
T = 2048
D = 256
H = 256

NC = 2
NS = 16
NW = NC * NS
CHUNK = 128
NQ = 4
LQ = L // NQ
SENTS = LQ // 2
WIN = 256
META = 16 + 2 * SENTS

CL = 16
NBLK = LQ // CL


def _sc_body(pv_hbm, meta_hbm, sidx_hbm, wemb_hbm, out_hbm,
             meta_v, idx_v, sidx_v, rows_v, outbuf_v, sem0, sem1, q):
    c = lax.axis_index("c")
    s = lax.axis_index("s")
    w = c * NS + s
    b = w // 2

    woff = pl.multiple_of(b * T, 8)
    pltpu.async_copy(pv_hbm.at[pl.ds(woff, T)], idx_v, sem1)
    pltpu.sync_copy(meta_hbm.at[q, w], meta_v)
    pltpu.sync_copy(sidx_hbm.at[w], sidx_v.at[0])
    hdr = meta_v[pl.ds(0, 16)]
    awstart = pl.multiple_of(hdr[0], 8)
    nch = hdr[1]
    pltpu.make_async_copy(pv_hbm.at[pl.ds(woff, T)], idx_v, sem1).wait()

    @pl.when(nch >= 1)
    def _():
        pltpu.async_copy(wemb_hbm.at[idx_v.at[pl.ds(awstart, CHUNK)]],
                         rows_v.at[pl.ds(0, CHUNK)], sem0)

    @pl.when(nch >= 2)
    def _():
        pltpu.async_copy(
            wemb_hbm.at[idx_v.at[pl.ds(awstart + CHUNK, CHUNK)]],
            rows_v.at[pl.ds(CHUNK, CHUNK)], sem1)

    @pl.when(nch >= 1)
    def _():
        pltpu.make_async_copy(wemb_hbm.at[idx_v.at[pl.ds(awstart, CHUNK)]],
                              rows_v.at[pl.ds(0, CHUNK)], sem0).wait()

    @pl.when(nch >= 2)
    def _():
        pltpu.make_async_copy(
            wemb_hbm.at[idx_v.at[pl.ds(awstart + CHUNK, CHUNK)]],
            rows_v.at[pl.ds(CHUNK, CHUNK)], sem1).wait()

    def group_body(g, carry):
        cs16 = meta_v[pl.ds(16 + g * 16, 16)]
        ln16 = meta_v[pl.ds(16 + SENTS + g * 16, 16)]
        for si in range(16):
            cs = cs16[si]
            ln = ln16[si]
            acc = [jnp.zeros((16,), jnp.float32) for _ in range(D // 16)]
            for k in range(SLOT_K):
                keep = k < ln
                for cg in range(D // 16):
                    v = rows_v[cs + k, pl.ds(cg * 16, 16)]
                    acc[cg] = acc[cg] + jnp.where(keep, v, 0.0)
            for cg in range(D // 16):
                outbuf_v[g * 16 + si, pl.ds(cg * 16, 16)] = acc[cg]
        return carry

    lax.fori_loop(0, SENTS // 16, group_body, 0)

    pltpu.async_copy(outbuf_v, out_hbm.at[sidx_v.at[0]], sem0).wait()


SLOT_K = 8


def _segment_sum_sc(q, pv, meta, sidx, W_emb):
    mesh = plsc.VectorSubcoreMesh(core_axis_name="c", subcore_axis_name="s")
    fn = pl.kernel(
        functools.partial(_sc_body, q=q),
        out_type=jax.ShapeDtypeStruct((LQ * B, D), jnp.float32),
        mesh=mesh,
        scratch_types=[
            pltpu.VMEM((META,), jnp.int32),
            pltpu.VMEM((T,), jnp.int32),
            pltpu.VMEM((1, SENTS), jnp.int32),
            pltpu.VMEM((WIN, D), jnp.float32),
            pltpu.VMEM((SENTS, D), jnp.float32),
            pltpu.SemaphoreType.DMA,
            pltpu.SemaphoreType.DMA,
        ],
    )
    return fn(pv, meta, sidx, W_emb)


DN = (((1,), (0,)), ((), ()))


def _lstm_body(toff, maxlen_ref, x_ref, wih_ref, whh_ref, bias_ref, len_ref,
               hin_ref, cin_ref, *rest):
    if len(rest) == 7:
        obuf_ref, out_ref, hout_ref, cout_ref, h_ref, c_ref, gx_ref = rest
    else:
        out_ref, hout_ref, cout_ref, h_ref, c_ref, gx_ref = rest
    blk = pl.program_id(0)

    @pl.when(blk == 0)
    def _():
        h_ref[...] = hin_ref[...]
        c_ref[...] = cin_ref[...]

    active = toff + blk * CL < maxlen_ref[0]

    @pl.when(active)
    def _():
        x2 = x_ref[...].reshape(CL * B, D).astype(jnp.bfloat16)
        gx_ref[...] = (
            lax.dot_general(x2, wih_ref[...], DN,
                            preferred_element_type=jnp.float32)
            + bias_ref[...]
        )

        lens = len_ref[...][:, 0:1]

        def step(j, carry):
            h, c = carry
            t = toff + blk * CL + j
            gates = gx_ref[pl.ds(j * B, B)] + lax.dot_general(
                h.astype(jnp.bfloat16), whh_ref[...], DN,
                preferred_element_type=jnp.float32)
            ii = jax.nn.sigmoid(gates[:, 0:H])
            ff = jax.nn.sigmoid(gates[:, H:2 * H])
            gg = jnp.tanh(gates[:, 2 * H:3 * H])
            oo = jax.nn.sigmoid(gates[:, 3 * H:4 * H])
            c_new = ff * c + ii * gg
            h_new = oo * jnp.tanh(c_new)
            mask = t < lens
            out_ref[j] = jnp.where(mask, h_new, 0.0)
            return (jnp.where(mask, h_new, h), jnp.where(mask, c_new, c))

        hh, cc = lax.fori_loop(0, CL, step, (h_ref[...], c_ref[...]),
                               unroll=16)
        h_ref[...] = hh
        c_ref[...] = cc

    @pl.when(jnp.logical_not(active))
    def _():
        out_ref[...] = jnp.zeros_like(out_ref)

    @pl.when(blk == NBLK - 1)
    def _():
        hout_ref[...] = h_ref[...]
        cout_ref[...] = c_ref[...]


def _lstm_tc(toff, maxlen, x, wih, whh, bias, lens, h0, c0, obuf):
    qbase = toff // CL
    in_specs = [
        pl.BlockSpec(memory_space=pltpu.SMEM),
        pl.BlockSpec((CL, B, D), lambda i: (i, 0, 0)),
        pl.BlockSpec((D, 4 * H), lambda i: (0, 0)),
        pl.BlockSpec((H, 4 * H), lambda i: (0, 0)),
        pl.BlockSpec((1, 4 * H), lambda i: (0, 0)),
        pl.BlockSpec((B, 128), lambda i: (0, 0)),
        pl.BlockSpec((B, H), lambda i: (0, 0)),
        pl.BlockSpec((B, H), lambda i: (0, 0)),
    ]
    args = [maxlen, x, wih, whh, bias, lens, h0, c0]
    aliases = {}
    if obuf is not None:
        in_specs.append(pl.BlockSpec(memory_space=pltpu.MemorySpace.HBM))
        args.append(obuf)
        aliases = {8: 0}
    return pl.pallas_call(
        functools.partial(_lstm_body, toff),
        grid=(NBLK,),
        in_specs=in_specs,
        out_specs=[
            pl.BlockSpec((CL, B, H), lambda i: (qbase + i, 0, 0)),
            pl.BlockSpec((B, H), lambda i: (0, 0)),
            pl.BlockSpec((B, H), lambda i: (0, 0)),
        ],
        out_shape=[
            jax.ShapeDtypeStruct((L, B, H), jnp.float32),
            jax.ShapeDtypeStruct((B, H), jnp.float32),
            jax.ShapeDtypeStruct((B, H), jnp.float32),
        ],
        scratch_shapes=[
            pltpu.VMEM((B, H), jnp.float32),
            pltpu.VMEM((B, H), jnp.float32),
            pltpu.VMEM((CL * B, 4 * H), jnp.float32),
        ],
        input_output_aliases=aliases,
    )(*args)


def kernel(paragraph_variable, sentence_length_list, paragh_length_list,
           max_no_lines, W_emb, W_ih, W_hh, b_ih, b_hh):
    sll = sentence_length_list.astype(jnp.int32)
    ends = jnp.cumsum(sll, axis=1)
    starts = ends - sll

    def per_worker(a):
        return jnp.transpose(
            a.reshape(B, NQ, 2, SENTS), (1, 0, 2, 3)).reshape(NQ, NW, SENTS)

    st_q = per_worker(starts)
    ln_q = per_worker(sll)
    end_q = per_worker(ends)

    awstart = (st_q[:, :, 0] // 8) * 8
    span = end_q[:, :, SENTS - 1] - awstart
    nch = (span + CHUNK - 1) // CHUNK
    cs = st_q - awstart[:, :, None]

    hdr = jnp.zeros((NQ, NW, 16), jnp.int32)
    hdr = hdr.at[:, :, 0].set(awstart).at[:, :, 1].set(nch)
    meta = jnp.concatenate([hdr, cs, ln_q], axis=2)

    hh_arr = jnp.arange(NW, dtype=jnp.int32) % 2
    b_of_w = jnp.arange(NW, dtype=jnp.int32) // 2
    j_arr = jnp.arange(SENTS, dtype=jnp.int32)
    sidx = (hh_arr[:, None] * SENTS + j_arr[None, :]) * B + b_of_w[:, None]

    pv = paragraph_variable.astype(jnp.int32).reshape(B * T)

    wih = W_ih.T.astype(jnp.bfloat16)
    whh = W_hh.T.astype(jnp.bfloat16)
    bias = (b_ih + b_hh).reshape(1, 4 * H)
    lens = jnp.broadcast_to(
        paragh_length_list.astype(jnp.int32)[:, None], (B, 128))
    maxlen = paragh_length_list.astype(jnp.int32)[:1]

    xs = [
        _segment_sum_sc(q, pv, meta, sidx, W_emb).reshape(LQ, B, D)
        for q in range(NQ)
    ]

    h = c = jnp.zeros((B, H), jnp.float32)
    out = None
    for q in range(NQ):
        out, h, c = _lstm_tc(q * LQ, maxlen, xs[q], wih, whh, bias,
                             lens, h, c, out)
    return out


# speedup vs baseline: 1.4653x; 1.2367x over previous
import functools

import jax
import jax.numpy as jnp
from jax import lax
from jax.experimental import pallas as pl
from jax.experimental.pallas import tpu as pltpu
from jax.experimental.pallas import tpu_sc as plsc

B = 16
L = 256
T = 2048
D = 256
H = 256

NC = 2
NS = 16
NW = NC * NS
CHUNK = 128
NQ = 4
LQ = L // NQ
SENTS = LQ // 2
WIN = 256
META = 16 + 2 * SENTS

CL = 16
NBLK = LQ // CL


def _sc_body(pv_hbm, meta_hbm, sidx_hbm, wemb_hbm, out_hbm,
             meta_v, idx_v, sidx_v, rows_v, outbuf_v, sem0, sem1, q):
    c = lax.axis_index("c")
    s = lax.axis_index("s")
    w = c * NS + s
    b = w // 2

    woff = pl.multiple_of(b * T, 8)
    pltpu.async_copy(pv_hbm.at[pl.ds(woff, T)], idx_v, sem1)
    pltpu.sync_copy(meta_hbm.at[q, w], meta_v)
    pltpu.sync_copy(sidx_hbm.at[w], sidx_v.at[0])
    hdr = meta_v[pl.ds(0, 16)]
    awstart = pl.multiple_of(hdr[0], 8)
    nch = hdr[1]
    pltpu.make_async_copy(pv_hbm.at[pl.ds(woff, T)], idx_v, sem1).wait()

    @pl.when(nch >= 1)
    def _():
        pltpu.async_copy(wemb_hbm.at[idx_v.at[pl.ds(awstart, CHUNK)]],
                         rows_v.at[pl.ds(0, CHUNK)], sem0)

    @pl.when(nch >= 2)
    def _():
        pltpu.async_copy(
            wemb_hbm.at[idx_v.at[pl.ds(awstart + CHUNK, CHUNK)]],
            rows_v.at[pl.ds(CHUNK, CHUNK)], sem1)

    @pl.when(nch >= 1)
    def _():
        pltpu.make_async_copy(wemb_hbm.at[idx_v.at[pl.ds(awstart, CHUNK)]],
                              rows_v.at[pl.ds(0, CHUNK)], sem0).wait()

    @pl.when(nch >= 2)
    def _():
        pltpu.make_async_copy(
            wemb_hbm.at[idx_v.at[pl.ds(awstart + CHUNK, CHUNK)]],
            rows_v.at[pl.ds(CHUNK, CHUNK)], sem1).wait()

    def group_body(g, carry):
        cs16 = meta_v[pl.ds(16 + g * 16, 16)]
        ln16 = meta_v[pl.ds(16 + SENTS + g * 16, 16)]
        for si in range(16):
            cs = cs16[si]
            ln = ln16[si]

            def tok_add(k, acc):
                return tuple(
                    acc[cg] + rows_v[cs + k, pl.ds(cg * 16, 16)]
                    for cg in range(D // 16))

            acc = lax.fori_loop(
                0, ln, tok_add,
                tuple(jnp.zeros((16,), jnp.float32)
                      for _ in range(D // 16)))
            for cg in range(D // 16):
                outbuf_v[g * 16 + si, pl.ds(cg * 16, 16)] = acc[cg]
        return carry

    lax.fori_loop(0, SENTS // 16, group_body, 0)

    pltpu.async_copy(outbuf_v, out_hbm.at[sidx_v.at[0]], sem0).wait()


SLOT_K = 8


def _segment_sum_sc(q, pv, meta, sidx, W_emb):
    mesh = plsc.VectorSubcoreMesh(core_axis_name="c", subcore_axis_name="s")
    fn = pl.kernel(
        functools.partial(_sc_body, q=q),
        out_type=jax.ShapeDtypeStruct((LQ * B, D), jnp.float32),
        mesh=mesh,
        scratch_types=[
            pltpu.VMEM((META,), jnp.int32),
            pltpu.VMEM((T,), jnp.int32),
            pltpu.VMEM((1, SENTS), jnp.int32),
            pltpu.VMEM((WIN, D), jnp.float32),
            pltpu.VMEM((SENTS, D), jnp.float32),
            pltpu.SemaphoreType.DMA,
            pltpu.SemaphoreType.DMA,
        ],
    )
    return fn(pv, meta, sidx, W_emb)


DN = (((1,), (0,)), ((), ()))


def _lstm_body(toff, maxlen_ref, x_ref, wih_ref, whh_ref, bias_ref, len_ref,
               hin_ref, cin_ref, *rest):
    if len(rest) == 7:
        obuf_ref, out_ref, hout_ref, cout_ref, h_ref, c_ref, gx_ref = rest
    else:
        out_ref, hout_ref, cout_ref, h_ref, c_ref, gx_ref = rest
    blk = pl.program_id(0)

    @pl.when(blk == 0)
    def _():
        h_ref[...] = hin_ref[...]
        c_ref[...] = cin_ref[...]

    active = toff + blk * CL < maxlen_ref[0]

    @pl.when(active)
    def _():
        x2 = x_ref[...].reshape(CL * B, D).astype(jnp.bfloat16)
        gx_ref[...] = (
            lax.dot_general(x2, wih_ref[...], DN,
                            preferred_element_type=jnp.float32)
            + bias_ref[...]
        )

        lens = len_ref[...][:, 0:1]

        def step(j, carry):
            h, c = carry
            t = toff + blk * CL + j
            gates = gx_ref[pl.ds(j * B, B)] + lax.dot_general(
                h.astype(jnp.bfloat16), whh_ref[...], DN,
                preferred_element_type=jnp.float32)
            ii = jax.nn.sigmoid(gates[:, 0:H])
            ff = jax.nn.sigmoid(gates[:, H:2 * H])
            gg = jnp.tanh(gates[:, 2 * H:3 * H])
            oo = jax.nn.sigmoid(gates[:, 3 * H:4 * H])
            c_new = ff * c + ii * gg
            h_new = oo * jnp.tanh(c_new)
            mask = t < lens
            out_ref[j] = jnp.where(mask, h_new, 0.0)
            return (jnp.where(mask, h_new, h), jnp.where(mask, c_new, c))

        hh, cc = lax.fori_loop(0, CL, step, (h_ref[...], c_ref[...]),
                               unroll=16)
        h_ref[...] = hh
        c_ref[...] = cc

    @pl.when(jnp.logical_not(active))
    def _():
        out_ref[...] = jnp.zeros_like(out_ref)

    @pl.when(blk == NBLK - 1)
    def _():
        hout_ref[...] = h_ref[...]
        cout_ref[...] = c_ref[...]


def _lstm_tc(toff, maxlen, x, wih, whh, bias, lens, h0, c0, obuf):
    qbase = toff // CL
    in_specs = [
        pl.BlockSpec(memory_space=pltpu.SMEM),
        pl.BlockSpec((CL, B, D), lambda i: (i, 0, 0)),
        pl.BlockSpec((D, 4 * H), lambda i: (0, 0)),
        pl.BlockSpec((H, 4 * H), lambda i: (0, 0)),
        pl.BlockSpec((1, 4 * H), lambda i: (0, 0)),
        pl.BlockSpec((B, 128), lambda i: (0, 0)),
        pl.BlockSpec((B, H), lambda i: (0, 0)),
        pl.BlockSpec((B, H), lambda i: (0, 0)),
    ]
    args = [maxlen, x, wih, whh, bias, lens, h0, c0]
    aliases = {}
    if obuf is not None:
        in_specs.append(pl.BlockSpec(memory_space=pltpu.MemorySpace.HBM))
        args.append(obuf)
        aliases = {8: 0}
    return pl.pallas_call(
        functools.partial(_lstm_body, toff),
        grid=(NBLK,),
        in_specs=in_specs,
        out_specs=[
            pl.BlockSpec((CL, B, H), lambda i: (qbase + i, 0, 0)),
            pl.BlockSpec((B, H), lambda i: (0, 0)),
            pl.BlockSpec((B, H), lambda i: (0, 0)),
        ],
        out_shape=[
            jax.ShapeDtypeStruct((L, B, H), jnp.float32),
            jax.ShapeDtypeStruct((B, H), jnp.float32),
            jax.ShapeDtypeStruct((B, H), jnp.float32),
        ],
        scratch_shapes=[
            pltpu.VMEM((B, H), jnp.float32),
            pltpu.VMEM((B, H), jnp.float32),
            pltpu.VMEM((CL * B, 4 * H), jnp.float32),
        ],
        input_output_aliases=aliases,
    )(*args)


def kernel(paragraph_variable, sentence_length_list, paragh_length_list,
           max_no_lines, W_emb, W_ih, W_hh, b_ih, b_hh):
    sll = sentence_length_list.astype(jnp.int32)
    ends = jnp.cumsum(sll, axis=1)
    starts = ends - sll

    def per_worker(a):
        return jnp.transpose(
            a.reshape(B, NQ, 2, SENTS), (1, 0, 2, 3)).reshape(NQ, NW, SENTS)

    st_q = per_worker(starts)
    ln_q = per_worker(sll)
    end_q = per_worker(ends)

    awstart = (st_q[:, :, 0] // 8) * 8
    span = end_q[:, :, SENTS - 1] - awstart
    nch = (span + CHUNK - 1) // CHUNK
    cs = st_q - awstart[:, :, None]

    hdr = jnp.zeros((NQ, NW, 16), jnp.int32)
    hdr = hdr.at[:, :, 0].set(awstart).at[:, :, 1].set(nch)
    meta = jnp.concatenate([hdr, cs, ln_q], axis=2)

    hh_arr = jnp.arange(NW, dtype=jnp.int32) % 2
    b_of_w = jnp.arange(NW, dtype=jnp.int32) // 2
    j_arr = jnp.arange(SENTS, dtype=jnp.int32)
    sidx = (hh_arr[:, None] * SENTS + j_arr[None, :]) * B + b_of_w[:, None]

    pv = paragraph_variable.astype(jnp.int32).reshape(B * T)

    wih = W_ih.T.astype(jnp.bfloat16)
    whh = W_hh.T.astype(jnp.bfloat16)
    bias = (b_ih + b_hh).reshape(1, 4 * H)
    lens = jnp.broadcast_to(
        paragh_length_list.astype(jnp.int32)[:, None], (B, 128))
    maxlen = paragh_length_list.astype(jnp.int32)[:1]

    xs = [
        _segment_sum_sc(q, pv, meta, sidx, W_emb).reshape(LQ, B, D)
        for q in range(NQ)
    ]

    h = c = jnp.zeros((B, H), jnp.float32)
    out = None
    for q in range(NQ):
        out, h, c = _lstm_tc(q * LQ, maxlen, xs[q], wih, whh, bias,
                             lens, h, c, out)
    return out
